# Initial kernel scaffold; baseline (speedup 1.0000x reference)
import jax
import jax.numpy as jnp
from jax.experimental import pallas as pl


def kernel(x, edge_index, batch, W_gat, att_src, att_dst, bias_gat, W_gate, b_gate, W_nn, b_nn, W_lin1, b_lin1, W_lin2, b_lin2):
    raise NotImplementedError("write your pallas kernel here")


# speedup vs baseline: 6.4453x; 6.4453x over previous
import functools

import jax
import jax.numpy as jnp
from jax import lax
from jax.experimental import pallas as pl
from jax.experimental.pallas import tpu as pltpu
from jax.experimental.pallas import tpu_sc as plsc

N_NODES = 10000
N_EDGES = 320000
NUM_GRAPHS = 64
IN_CH = 128
HID = 512
NP = 10240
BN = 1280
NBLK = NP // BN
NC, NS, L = 2, 16, 16
NW = NC * NS
EPT = 10368
EP = EPT * NW
GE = 128
NG = EPT // GE
NB = 3
STRIPE = NP // NS
NQ = 16
QW = HID // NQ
NEG = -3.0e38


def _k1_body(x_ref, w_ref, atts_ref, attd_ref, h4_ref, asrc_ref, adst_ref,
             bmax_ref):
    i = pl.program_id(0)
    h = jnp.dot(x_ref[...], w_ref[...], preferred_element_type=jnp.float32)
    a_s = jnp.sum(h * atts_ref[...], axis=1)
    a_d = jnp.sum(h * attd_ref[...], axis=1)
    rows = i * BN + lax.broadcasted_iota(jnp.int32, (BN,), 0)
    a_s = jnp.where(rows < N_NODES, a_s, NEG)
    a_d = jnp.where(rows < N_NODES, a_d, NEG)
    for q in range(NQ):
        h4_ref[q] = h[:, q * QW:(q + 1) * QW]
    asrc_ref[0, 0, :] = a_s
    adst_ref[0, 0, :] = a_d
    bmax_ref[...] = jnp.max(a_s).reshape(1, 1, 1)


def _k1(x_p, w_gat, att_src, att_dst):
    return pl.pallas_call(
        _k1_body,
        grid=(NBLK,),
        in_specs=[
            pl.BlockSpec((BN, IN_CH), lambda i: (i, 0)),
            pl.BlockSpec((IN_CH, HID), lambda i: (0, 0)),
            pl.BlockSpec((1, HID), lambda i: (0, 0)),
            pl.BlockSpec((1, HID), lambda i: (0, 0)),
        ],
        out_specs=[
            pl.BlockSpec((NQ, BN, QW), lambda i: (0, i, 0)),
            pl.BlockSpec((1, 1, BN), lambda i: (i, 0, 0)),
            pl.BlockSpec((1, 1, BN), lambda i: (i, 0, 0)),
            pl.BlockSpec((1, 1, 1), lambda i: (i, 0, 0)),
        ],
        out_shape=[
            jax.ShapeDtypeStruct((NQ, NP, QW), jnp.float32),
            jax.ShapeDtypeStruct((NBLK, 1, BN), jnp.float32),
            jax.ShapeDtypeStruct((NBLK, 1, BN), jnp.float32),
            jax.ShapeDtypeStruct((NBLK, 1, 1), jnp.float32),
        ],
    )(x_p, w_gat, att_src.reshape(1, HID), att_dst.reshape(1, HID))


def _k1b_body(bmax_ref, adst_ref, m_ref):
    max_a = jnp.max(bmax_ref[...])
    m = max_a + adst_ref[...]
    m_ref[...] = jnp.where(m >= 0, m, 0.2 * m)


def _k1b(bmax, adst2):
    return pl.pallas_call(
        _k1b_body,
        out_shape=jax.ShapeDtypeStruct((NBLK, 1, BN), jnp.float32),
    )(bmax, adst2)


def _k2_body(src_hbm, dst_hbm, asrc_hbm, adst_hbm, m_hbm, h4_hbm,
             out_hbm, s_hbm,
             src_v, dst_v, e_v, asrc_v, adst_v, m_v, rows, s_buf,
             out_sh, g0, g1, g2, t0, t1, t2):
    c = lax.axis_index("c")
    sid = lax.axis_index("s")
    wid = c * NS + sid
    gsem = (g0, g1, g2)
    tsem = (t0, t1, t2)

    pltpu.sync_copy(src_hbm.at[wid], src_v)
    pltpu.sync_copy(dst_hbm.at[wid], dst_v)
    pltpu.sync_copy(asrc_hbm, asrc_v)
    pltpu.sync_copy(adst_hbm, adst_v)
    pltpu.sync_copy(m_hbm, m_v)

    zero16 = jnp.zeros((L,), jnp.float32)

    def _zs(k, _):
        for ch in range(QW // L):
            s_buf[k, pl.ds(ch * L, L)] = zero16
        return 0
    lax.fori_loop(0, GE, _zs, 0)

    def _e_of(g, _):
        for j in range(GE // L):
            sl = pl.ds(j * L, L)
            si = src_v[g, sl]
            di = dst_v[g, sl]
            a1 = plsc.load_gather(asrc_v, [si])
            a2 = plsc.load_gather(adst_v, [di])
            mm = plsc.load_gather(m_v, [di])
            sc = a1 + a2
            sc = jnp.where(sc >= 0, sc, 0.2 * sc)
            e_v[pl.ds(g * GE + j * L, L)] = jnp.exp(sc - mm)
        return 0
    lax.fori_loop(0, NG, _e_of, 0)

    row0 = sid * STRIPE
    lane = lax.iota(jnp.int32, L)

    def _col_pass(q, _):
        def _zr(k, _):
            for ch in range(QW // L):
                rows[0, k, pl.ds(ch * L, L)] = zero16
            return 0
        lax.fori_loop(0, GE, _zr, 0)
        for t in range(STRIPE // GE):
            pltpu.sync_copy(rows.at[0], out_sh.at[pl.ds(row0 + t * GE, GE)])
        plsc.subcore_barrier()

        def _super(gb, _):
            for b in range(NB):
                g = gb * NB + b
                pltpu.async_copy(h4_hbm.at[q].at[src_v.at[g]], rows.at[b],
                                 gsem[b])
            for b in range(NB):
                g = gb * NB + b
                pltpu.make_async_copy(h4_hbm.at[q].at[src_v.at[g]],
                                      rows.at[b], gsem[b]).wait()

                def _scale(k, _):
                    idx = jnp.zeros((L,), jnp.int32) + (g * GE + k)
                    ev = plsc.load_gather(e_v, [idx])
                    for ch in range(QW // L):
                        sl = pl.ds(ch * L, L)
                        rows[b, k, sl] = rows[b, k, sl] * ev
                    return 0
                lax.fori_loop(0, GE, _scale, 0)

                pltpu.async_copy(rows.at[b], out_sh.at[dst_v.at[g]],
                                 tsem[b], add=True)
            for b in range(NB):
                g = gb * NB + b
                pltpu.make_async_copy(rows.at[b], out_sh.at[dst_v.at[g]],
                                      tsem[b]).wait()
            return 0
        lax.fori_loop(0, NG // NB, _super, 0)

        plsc.subcore_barrier()
        for t in range(STRIPE // GE):
            sl = pl.ds(row0 + t * GE, GE)
            pltpu.sync_copy(out_sh.at[sl], out_hbm.at[c, q].at[sl])
        plsc.subcore_barrier()
        return 0
    lax.fori_loop(0, NQ, _col_pass, 0)

    for t in range(STRIPE // GE):
        pltpu.sync_copy(s_buf, out_sh.at[pl.ds(row0 + t * GE, GE)])
    plsc.subcore_barrier()

    def _s_pass(g, _):
        for j in range(GE // L):
            ridx = lane + j * L
            cidx = jnp.zeros((L,), jnp.int32)
            plsc.store_scatter(s_buf, [ridx, cidx],
                               e_v[pl.ds(g * GE + j * L, L)])
        pltpu.sync_copy(s_buf, out_sh.at[dst_v.at[g]], add=True)
        return 0
    lax.fori_loop(0, NG, _s_pass, 0)

    plsc.subcore_barrier()
    for t in range(STRIPE // GE):
        sl = pl.ds(row0 + t * GE, GE)
        pltpu.sync_copy(out_sh.at[sl], s_hbm.at[c].at[sl])


def _k2(src_p, dst_p, asrc, adst, m, h4):
    mesh = plsc.VectorSubcoreMesh(core_axis_name="c", subcore_axis_name="s",
                                  num_cores=NC, num_subcores=NS)
    f = functools.partial(
        pl.kernel,
        out_type=(jax.ShapeDtypeStruct((NC, NQ, NP, QW), jnp.float32),
                  jax.ShapeDtypeStruct((NC, NP, QW), jnp.float32)),
        mesh=mesh,
        compiler_params=pltpu.CompilerParams(needs_layout_passes=False,
                                             use_tc_tiling_on_sc=False),
        scratch_types=[
            pltpu.VMEM((NG, GE), jnp.int32),
            pltpu.VMEM((NG, GE), jnp.int32),
            pltpu.VMEM((EPT,), jnp.float32),
            pltpu.VMEM((NP,), jnp.float32),
            pltpu.VMEM((NP,), jnp.float32),
            pltpu.VMEM((NP,), jnp.float32),
            pltpu.VMEM((NB, GE, QW), jnp.float32),
            pltpu.VMEM((GE, QW), jnp.float32),
            pltpu.VMEM_SHARED((NP, QW), jnp.float32),
            pltpu.SemaphoreType.DMA,
            pltpu.SemaphoreType.DMA,
            pltpu.SemaphoreType.DMA,
            pltpu.SemaphoreType.DMA,
            pltpu.SemaphoreType.DMA,
            pltpu.SemaphoreType.DMA,
        ],
    )(_k2_body)
    return f(src_p, dst_p, asrc, adst, m, h4)


def _k3a_body(p_ref, s_ref, bias_ref, wg_ref, bg_ref, out_ref, gate_ref):
    p = p_ref[...]
    s = jnp.sum(s_ref[...], axis=(0, 2)) + 1e-16
    acc = p[0] + p[1]
    full = jnp.concatenate([acc[q] for q in range(NQ)], axis=1)
    out = full / s[:, None] + bias_ref[...]
    out_ref[...] = out
    gate_ref[0, 0, :] = jnp.sum(out * wg_ref[...], axis=1) + bg_ref[0]


def _k3a(parts, s_parts, bias_gat, w_gate, b_gate):
    return pl.pallas_call(
        _k3a_body,
        grid=(NBLK,),
        in_specs=[
            pl.BlockSpec((NC, NQ, BN, QW), lambda i: (0, 0, i, 0)),
            pl.BlockSpec((NC, BN, QW), lambda i: (0, i, 0)),
            pl.BlockSpec((1, HID), lambda i: (0, 0)),
            pl.BlockSpec((1, HID), lambda i: (0, 0)),
            pl.BlockSpec((1, 1), lambda i: (0, 0)),
        ],
        out_specs=[
            pl.BlockSpec((BN, HID), lambda i: (i, 0)),
            pl.BlockSpec((1, 1, BN), lambda i: (i, 0, 0)),
        ],
        out_shape=[
            jax.ShapeDtypeStruct((NP, HID), jnp.float32),
            jax.ShapeDtypeStruct((NBLK, 1, BN), jnp.float32),
        ],
    )(parts, s_parts, bias_gat.reshape(1, HID), w_gate.reshape(1, HID),
      b_gate.reshape(1, 1))


def _k3b_body(gate_ref, batch_ref, ms_ref, gsm_ref):
    gate = gate_ref[:, 0, :]
    batch = batch_ref[:, 0, :]
    gids = lax.broadcasted_iota(jnp.int32, (NUM_GRAPHS, BN), 0)
    m_g = jnp.full((NUM_GRAPHS, 1), NEG, jnp.float32)
    for r in range(NBLK):
        masked = jnp.where(batch[r:r + 1, :] == gids, gate[r:r + 1, :], NEG)
        m_g = jnp.maximum(m_g, jnp.max(masked, axis=1, keepdims=True))
    s_g = jnp.zeros((NUM_GRAPHS, 1), jnp.float32)
    for r in range(NBLK):
        oh = (batch[r:r + 1, :] == gids).astype(jnp.float32)
        m_r = jnp.sum(oh * m_g, axis=0, keepdims=True)
        e_r = jnp.exp(gate[r:r + 1, :] - m_r)
        s_g = s_g + jnp.sum(oh * e_r, axis=1, keepdims=True)
        gsm_ref[r, 0, :] = e_r[0, :]
    ms_ref[...] = jnp.concatenate([m_g, s_g], axis=1)
    for r in range(NBLK):
        oh = (batch[r:r + 1, :] == gids).astype(jnp.float32)
        s_r = jnp.sum(oh * s_g, axis=0, keepdims=True)
        gsm_ref[r, 0, :] = gsm_ref[r, 0, :] / (s_r[0, :] + 1e-16)


def _k3b(gate2, batch2):
    return pl.pallas_call(
        _k3b_body,
        out_shape=[
            jax.ShapeDtypeStruct((NUM_GRAPHS, 2), jnp.float32),
            jax.ShapeDtypeStruct((NBLK, 1, BN), jnp.float32),
        ],
    )(gate2, batch2)


def _k3c_body(out_ref, batch_ref, gsm_ref, wnn_ref, bnn_ref, w1_ref, b1_ref,
              w2_ref, b2_ref, fin_ref, res_ref):
    i = pl.program_id(0)

    @pl.when(i == 0)
    def _():
        res_ref[...] = jnp.zeros((NUM_GRAPHS, HID), jnp.float32)

    xn = jnp.dot(out_ref[...], wnn_ref[...],
                 preferred_element_type=jnp.float32) + bnn_ref[...]
    gids = lax.broadcasted_iota(jnp.int32, (NUM_GRAPHS, BN), 0)
    oh = (batch_ref[0] == gids).astype(jnp.float32) * gsm_ref[0]
    res_ref[...] += jnp.dot(oh, xn, preferred_element_type=jnp.float32)

    @pl.when(i == NBLK - 1)
    def _():
        z = jnp.dot(res_ref[...], w1_ref[...],
                    preferred_element_type=jnp.float32) + b1_ref[...]
        z = jnp.maximum(z, 0.0)
        zz = jnp.dot(z, w2_ref[...],
                     preferred_element_type=jnp.float32) + b2_ref[...]
        fin_ref[...] = 1.0 / (1.0 + jnp.exp(-zz))


def _k3c(out_n, batch2, gsm2, w_nn, b_nn, w_lin1, b_lin1, w_lin2, b_lin2):
    return pl.pallas_call(
        _k3c_body,
        grid=(NBLK,),
        in_specs=[
            pl.BlockSpec((BN, HID), lambda i: (i, 0)),
            pl.BlockSpec((1, 1, BN), lambda i: (i, 0, 0)),
            pl.BlockSpec((1, 1, BN), lambda i: (i, 0, 0)),
            pl.BlockSpec((HID, HID), lambda i: (0, 0)),
            pl.BlockSpec((1, HID), lambda i: (0, 0)),
            pl.BlockSpec((HID, HID), lambda i: (0, 0)),
            pl.BlockSpec((1, HID), lambda i: (0, 0)),
            pl.BlockSpec((HID, 1), lambda i: (0, 0)),
            pl.BlockSpec((1, 1), lambda i: (0, 0)),
        ],
        out_specs=pl.BlockSpec((NUM_GRAPHS, 1), lambda i: (0, 0)),
        out_shape=jax.ShapeDtypeStruct((NUM_GRAPHS, 1), jnp.float32),
        scratch_shapes=[pltpu.VMEM((NUM_GRAPHS, HID), jnp.float32)],
    )(out_n, batch2, gsm2, w_nn, b_nn.reshape(1, HID), w_lin1,
      b_lin1.reshape(1, HID), w_lin2, b_lin2.reshape(1, 1))


def kernel(x, edge_index, batch, W_gat, att_src, att_dst, bias_gat,
           W_gate, b_gate, W_nn, b_nn, W_lin1, b_lin1, W_lin2, b_lin2):
    n = x.shape[0]
    x_p = jnp.pad(x, ((0, NP - n), (0, 0)))
    loop = jnp.arange(n, dtype=jnp.int32)
    pad_e = jnp.full((EP - N_EDGES - n,), N_NODES, jnp.int32)
    src_p = jnp.concatenate([edge_index[0].astype(jnp.int32), loop, pad_e])
    dst_p = jnp.concatenate([edge_index[1].astype(jnp.int32), loop, pad_e])
    src_p = src_p.reshape(NW, NG, GE)
    dst_p = dst_p.reshape(NW, NG, GE)
    batch_p = jnp.concatenate(
        [batch.astype(jnp.int32),
         jnp.full((NP - n,), NUM_GRAPHS, jnp.int32)]).reshape(NBLK, 1, BN)

    h4, asrc2, adst2, bmax = _k1(x_p, W_gat, att_src, att_dst)
    m2 = _k1b(bmax, adst2)
    parts, s_parts = _k2(src_p, dst_p, asrc2.reshape(NP), adst2.reshape(NP),
                         m2.reshape(NP), h4)
    out_n, gate2 = _k3a(parts, s_parts, bias_gat, W_gate, b_gate)
    ms, gsm2 = _k3b(gate2, batch_p)
    del ms
    fin = _k3c(out_n, batch_p, gsm2, W_nn, b_nn,
               W_lin1, b_lin1, W_lin2, b_lin2)
    out_final = fin.reshape(NUM_GRAPHS)
    gate_sm = gsm2.reshape(NP)[:n][:, None]
    return (out_final, gate_sm)
